# baseline (speedup 1.0000x reference)
import functools

import jax
import jax.numpy as jnp
from jax import lax
from jax.experimental import pallas as pl
from jax.experimental.pallas import tpu as pltpu
from jax.experimental.pallas import tpu_sc as plsc

B = 16384
F = 26
V = 100000
D = 16
ND = 13
H1 = 256
H2 = 128
FD = F * D

_NC = 2
_NS = 16
_NW = _NC * _NS
_BF = B * F
_R = _BF // _NW
_C = _R // 2


def _sc_gather_body(emb_hbm, lin_hbm, idx_hbm, out_emb, out_lin,
                    idx_v, rows_v, lin_v, sem_e, sem_l):
    wid = lax.axis_index("s") * _NC + lax.axis_index("c")
    base = wid * _R
    for it in range(_R // _C):
        off = base + it * _C
        pltpu.sync_copy(idx_hbm.at[pl.ds(off, _C)], idx_v)
        ge = pltpu.async_copy(emb_hbm.at[idx_v], rows_v, sem_e)
        gl = pltpu.async_copy(lin_hbm.at[idx_v], lin_v, sem_l)
        ge.wait()
        gl.wait()
        pltpu.sync_copy(rows_v, out_emb.at[pl.ds(off, _C)])
        pltpu.sync_copy(lin_v, out_lin.at[pl.ds(off, _C)])


def _sc_gather(emb_flat, lin_flat, flat_idx):
    call = pl.kernel(
        _sc_gather_body,
        out_type=(
            jax.ShapeDtypeStruct((_BF, D), jnp.float32),
            jax.ShapeDtypeStruct((_BF,), jnp.float32),
        ),
        mesh=plsc.VectorSubcoreMesh(core_axis_name="c", subcore_axis_name="s"),
        scratch_types=[
            pltpu.VMEM((_C,), jnp.int32),
            pltpu.VMEM((_C, D), jnp.float32),
            pltpu.VMEM((_C,), jnp.float32),
            pltpu.SemaphoreType.DMA,
            pltpu.SemaphoreType.DMA,
        ],
        compiler_params=pltpu.CompilerParams(use_tc_tiling_on_sc=False),
    )
    return call(emb_flat, lin_flat, flat_idx)


_BB = 1024


def _tc_body(fen_ref, lin_ref, den_ref, w1_ref, b1_ref, w2_ref, b2_ref,
             p_ref, e_ref, s_ref, dw_ref, bias_ref, out_ref):
    fen = fen_ref[...]
    h = jnp.dot(fen, w1_ref[...], preferred_element_type=jnp.float32)
    h = jnp.maximum(h + b1_ref[...], 0.0)
    h = jnp.dot(h, w2_ref[...], preferred_element_type=jnp.float32)
    h = jnp.maximum(h + b2_ref[...], 0.0)
    logits = jnp.dot(h, p_ref[...], preferred_element_type=jnp.float32)
    col = lax.broadcasted_iota(jnp.int32, logits.shape, 1)
    logits = jnp.where(col < F, logits, -jnp.inf)
    m = jnp.max(logits, axis=1, keepdims=True)
    e = jnp.exp(logits - m)
    mx = (float(F) / jnp.sum(e, axis=1, keepdims=True)) * e
    mx_exp = jnp.dot(mx, e_ref[...], preferred_element_type=jnp.float32)
    v = mx_exp * fen
    sv = jnp.dot(v, s_ref[...], preferred_element_type=jnp.float32)
    fm = 0.5 * (jnp.sum(sv * sv, axis=1) - jnp.sum(v * v, axis=1))
    sp = jnp.sum(lin_ref[...] * mx[:, :F], axis=1)
    dn = jnp.sum(den_ref[...] * dw_ref[...], axis=1)
    out_ref[...] = fm + sp + dn + bias_ref[0, 0]


def kernel(sparse, dense, emb, lin_w, dense_w, W1, b1, W2, b2, P, bias):
    emb_flat = emb.reshape(F * V, D)
    lin_flat = lin_w.reshape(F * V)
    flat_idx = (sparse + (jnp.arange(F, dtype=jnp.int32) * V)[None, :]).reshape(_BF)

    gath_emb, gath_lin = _sc_gather(emb_flat, lin_flat, flat_idx)
    fen = gath_emb.reshape(B, FD)
    lin2 = gath_lin.reshape(B, F)

    expand = jnp.zeros((H2, FD), jnp.float32).at[:F].set(
        jnp.repeat(jnp.eye(F, dtype=jnp.float32), D, axis=1))
    reduce_ = jnp.zeros((FD, H2), jnp.float32).at[:, :D].set(
        jnp.tile(jnp.eye(D, dtype=jnp.float32), (F, 1)))
    p_pad = jnp.zeros((H2, H2), jnp.float32).at[:, :F].set(P)

    out = pl.pallas_call(
        _tc_body,
        grid=(B // _BB,),
        in_specs=[
            pl.BlockSpec((_BB, FD), lambda i: (i, 0)),
            pl.BlockSpec((_BB, F), lambda i: (i, 0)),
            pl.BlockSpec((_BB, ND), lambda i: (i, 0)),
            pl.BlockSpec((FD, H1), lambda i: (0, 0)),
            pl.BlockSpec((1, H1), lambda i: (0, 0)),
            pl.BlockSpec((H1, H2), lambda i: (0, 0)),
            pl.BlockSpec((1, H2), lambda i: (0, 0)),
            pl.BlockSpec((H2, H2), lambda i: (0, 0)),
            pl.BlockSpec((H2, FD), lambda i: (0, 0)),
            pl.BlockSpec((FD, H2), lambda i: (0, 0)),
            pl.BlockSpec((1, ND), lambda i: (0, 0)),
            pl.BlockSpec((1, 1), lambda i: (0, 0)),
        ],
        out_specs=pl.BlockSpec((_BB,), lambda i: (i,)),
        out_shape=jax.ShapeDtypeStruct((B,), jnp.float32),
    )(fen, lin2, dense, W1, b1.reshape(1, H1), W2, b2.reshape(1, H2),
      p_pad, expand, reduce_, dense_w.reshape(1, ND), bias.reshape(1, 1))
    return out


# speedup vs baseline: 2.1922x; 2.1922x over previous
import functools

import jax
import jax.numpy as jnp
from jax import lax
from jax.experimental import pallas as pl
from jax.experimental.pallas import tpu as pltpu
from jax.experimental.pallas import tpu_sc as plsc

B = 16384
F = 26
V = 100000
D = 16
ND = 13
H1 = 256
H2 = 128
FD = F * D

_NC = 2
_NS = 16
_NW = _NC * _NS
_NTASK = FD + F
_NROUND = -(-_NTASK // _NW)
_CH = 4096


def _sc_gather_body(emb_hbm, lin_hbm, idx_hbm, fen_out, lin_out,
                    row_v, idx_v, res_v):
    wid = lax.axis_index("s") * _NC + lax.axis_index("c")

    def do_task(t):
        is_lin = t >= FD
        f = jnp.where(is_lin, t - FD, t // D)
        pltpu.sync_copy(idx_hbm.at[pl.ds(f * B, B)], idx_v)

        @pl.when(jnp.logical_not(is_lin))
        def _():
            pltpu.sync_copy(emb_hbm.at[pl.ds(t * V, V)], row_v)

        @pl.when(is_lin)
        def _():
            pltpu.sync_copy(lin_hbm.at[pl.ds((t - FD) * V, V)], row_v)

        def chunk(cix, carry):
            def step(i, c2):
                ii = idx_v[pl.ds(cix * _CH + i * 16, 16)]
                res_v[pl.ds(i * 16, 16)] = plsc.load_gather(row_v, [ii])
                return c2

            lax.fori_loop(0, _CH // 16, step, 0, unroll=4)

            @pl.when(jnp.logical_not(is_lin))
            def _store_fen():
                pltpu.sync_copy(
                    res_v, fen_out.at[pl.ds(t * B + cix * _CH, _CH)])

            @pl.when(is_lin)
            def _store_lin():
                pltpu.sync_copy(
                    res_v, lin_out.at[pl.ds((t - FD) * B + cix * _CH, _CH)])

            return carry

        lax.fori_loop(0, B // _CH, chunk, 0)

    for j in range(_NROUND):
        t = wid + _NW * j
        if (j + 1) * _NW <= _NTASK:
            do_task(t)
        else:
            @pl.when(t < _NTASK)
            def _():
                do_task(t)


def _sc_gather(emb_flat, lin_flat, idx_flat):
    call = pl.kernel(
        _sc_gather_body,
        out_type=(
            jax.ShapeDtypeStruct((FD * B,), jnp.float32),
            jax.ShapeDtypeStruct((F * B,), jnp.float32),
        ),
        mesh=plsc.VectorSubcoreMesh(core_axis_name="c", subcore_axis_name="s"),
        scratch_types=[
            pltpu.VMEM((V,), jnp.float32),
            pltpu.VMEM((B,), jnp.int32),
            pltpu.VMEM((_CH,), jnp.float32),
        ],
        compiler_params=pltpu.CompilerParams(
            use_tc_tiling_on_sc=False, needs_layout_passes=False),
    )
    return call(emb_flat, lin_flat, idx_flat)


_BB = 2048


def _tc_body(fen_ref, lin_ref, den_ref, w1t_ref, b1_ref, w2t_ref, b2_ref,
             pt_ref, e_ref, s_ref, dw_ref, bias_ref, out_ref):
    x = fen_ref[...]
    h = jnp.dot(w1t_ref[...], x, preferred_element_type=jnp.float32)
    h = jnp.maximum(h + b1_ref[...], 0.0)
    h = jnp.dot(w2t_ref[...], h, preferred_element_type=jnp.float32)
    h = jnp.maximum(h + b2_ref[...], 0.0)
    logits = jnp.dot(pt_ref[...], h, preferred_element_type=jnp.float32)
    m = jnp.max(logits, axis=0, keepdims=True)
    e = jnp.exp(logits - m)
    mx = (float(F) / jnp.sum(e, axis=0, keepdims=True)) * e
    mx_exp = jnp.dot(e_ref[...], mx, preferred_element_type=jnp.float32)
    v = mx_exp * x
    sv = jnp.dot(s_ref[...], v, preferred_element_type=jnp.float32)
    fm = 0.5 * (jnp.sum(sv * sv, axis=0) - jnp.sum(v * v, axis=0))
    sp = jnp.sum(lin_ref[...] * mx, axis=0)
    dn = jnp.sum(den_ref[...] * dw_ref[...], axis=0)
    out_ref[...] = fm + sp + dn + bias_ref[0, 0]


def kernel(sparse, dense, emb, lin_w, dense_w, W1, b1, W2, b2, P, bias):
    emb_flat = emb.transpose(0, 2, 1).reshape(FD * V)
    lin_flat = lin_w.reshape(F * V)
    idx_flat = sparse.T.reshape(F * B)

    fen_flat, lin_g = _sc_gather(emb_flat, lin_flat, idx_flat)
    fen_t = fen_flat.reshape(FD, B)
    lin_t = lin_g.reshape(F, B)
    dense_t = dense.T

    expand_t = jnp.repeat(jnp.eye(F, dtype=jnp.float32), D, axis=0)
    reduce_t = jnp.tile(jnp.eye(D, dtype=jnp.float32), (1, F))

    out = pl.pallas_call(
        _tc_body,
        grid=(B // _BB,),
        in_specs=[
            pl.BlockSpec((FD, _BB), lambda i: (0, i)),
            pl.BlockSpec((F, _BB), lambda i: (0, i)),
            pl.BlockSpec((ND, _BB), lambda i: (0, i)),
            pl.BlockSpec((H1, FD), lambda i: (0, 0)),
            pl.BlockSpec((H1, 1), lambda i: (0, 0)),
            pl.BlockSpec((H2, H1), lambda i: (0, 0)),
            pl.BlockSpec((H2, 1), lambda i: (0, 0)),
            pl.BlockSpec((F, H2), lambda i: (0, 0)),
            pl.BlockSpec((FD, F), lambda i: (0, 0)),
            pl.BlockSpec((D, FD), lambda i: (0, 0)),
            pl.BlockSpec((ND, 1), lambda i: (0, 0)),
            pl.BlockSpec((1, 1), lambda i: (0, 0)),
        ],
        out_specs=pl.BlockSpec((_BB,), lambda i: (i,)),
        out_shape=jax.ShapeDtypeStruct((B,), jnp.float32),
    )(fen_t, lin_t, dense_t, W1.T, b1.reshape(H1, 1), W2.T, b2.reshape(H2, 1),
      P.T, expand_t, reduce_t, dense_w.reshape(ND, 1), bias.reshape(1, 1))
    return out
